# baseline (speedup 1.0000x reference)
import jax
import jax.numpy as jnp
from jax.experimental import pallas as pl
from jax.experimental.pallas import tpu as pltpu


def _body(x_ref, g_ref, a_ref, o_ref):
    s1 = jnp.sum(x_ref[...], axis=1, keepdims=True)
    s2 = jnp.sum(g_ref[...], axis=1, keepdims=True)
    s3 = jnp.sum(a_ref[...], axis=1, keepdims=True)
    o_ref[...] = s3[0:o_ref.shape[0]] + jnp.sum(s1) + jnp.sum(s2)


def kernel(seq1, adj, glob_neg_seq, glob_neg_adj, alpha, W_fc, gcn_bias,
           prelu_a, W_bil, b_bil):
    B, N, N_IN = seq1.shape
    N_H = 64
    BB = 2000
    G = B // BB
    x2 = seq1.reshape(2 * B, 2 * N_IN)
    g2 = glob_neg_seq.reshape(2 * B, 2 * N_H)
    a2 = adj.transpose(0, 2, 1).reshape(2 * B, 8)
    l1 = pl.pallas_call(
        _body,
        grid=(G,),
        in_specs=[
            pl.BlockSpec((2 * BB, 2 * N_IN), lambda i: (i, 0)),
            pl.BlockSpec((2 * BB, 2 * N_H), lambda i: (i, 0)),
            pl.BlockSpec((2 * BB, 8), lambda i: (i, 0)),
        ],
        out_specs=pl.BlockSpec((BB, 1), lambda i: (i, 0)),
        out_shape=jax.ShapeDtypeStruct((B, 1), jnp.float32),
        compiler_params=pltpu.CompilerParams(
            dimension_semantics=("arbitrary",),
        ),
    )(x2, g2, a2)
    return jnp.concatenate([l1, l1], axis=0)


# speedup vs baseline: 1.9136x; 1.9136x over previous
import numpy as np

import jax
import jax.numpy as jnp
from jax.experimental import pallas as pl
from jax.experimental.pallas import tpu as pltpu


def _body(x_ref, adj_ref, gns_ref, wfc_ref, q_ref, wbil_ref,
          bias_ref, al_ref, pa_ref, bb_ref,
          l1_ref, l2_ref, fix_ref,
          gprev_ref, t0_ref):
    i = pl.program_id(0)
    nblk = pl.num_programs(0)

    n_h = wbil_ref.shape[0]
    a = pa_ref[0, 0]
    al = al_ref[0, 0]
    bb = bb_ref[0, 0]
    bias = bias_ref[...]

    fts_flat = jnp.dot(x_ref[...], wfc_ref[...],
                       preferred_element_type=jnp.float32)
    bbk = fts_flat.shape[0] // 4
    fts3 = fts_flat.reshape(bbk, 4, n_h)
    f0 = fts3[:, 0, :]
    f1 = fts3[:, 1, :]
    f2 = fts3[:, 2, :]
    f3 = fts3[:, 3, :]

    abig = jnp.dot(adj_ref[...], q_ref[...],
                   preferred_element_type=jnp.float32)

    def node(r):
        o = (abig[:, 256 * r + 0:256 * r + 64] * f0
             + abig[:, 256 * r + 64:256 * r + 128] * f1
             + abig[:, 256 * r + 128:256 * r + 192] * f2
             + abig[:, 256 * r + 192:256 * r + 256] * f3) + bias
        return jnp.where(o >= 0.0, o, a * o)

    c = (node(0) + node(1) + node(2)) * (1.0 / 3.0)
    hmv = node(3)

    gns = gns_ref[...]
    negc = (gns[:, 0:n_h] + gns[:, n_h:2 * n_h]
            + gns[:, 2 * n_h:3 * n_h]) * (1.0 / 3.0)

    g = al * c + (1.0 - al) * negc
    t = jnp.dot(hmv, wbil_ref[...], preferred_element_type=jnp.float32)

    l1_ref[...] = jnp.sum(t * g, axis=1, keepdims=True) + bb

    g_roll = pltpu.roll(g, 1, axis=0)
    row0 = jax.lax.broadcasted_iota(jnp.int32, g.shape, 0) == 0
    g_sh = jnp.where(row0, gprev_ref[...], g_roll)
    l2_ref[...] = jnp.sum(t * g_sh, axis=1, keepdims=True) + bb

    gprev_ref[...] = g[bbk - 1:bbk, :]

    @pl.when(i == 0)
    def _():
        t0_ref[...] = t[0:1, :]

    @pl.when(i == nblk - 1)
    def _():
        fix_ref[...] = jnp.sum(t0_ref[...] * g[bbk - 2:bbk - 1, :],
                               axis=1, keepdims=True) + bb


def kernel(seq1, adj, glob_neg_seq, glob_neg_adj, alpha, W_fc, gcn_bias,
           prelu_a, W_bil, b_bil):
    B, N, N_IN = seq1.shape
    N_H = W_fc.shape[1]
    BB = 2000
    G = B // BB

    x = seq1.reshape(B * N, N_IN)
    adj2 = adj.reshape(B, N * N)
    gns = glob_neg_seq.reshape(B, N * N_H)
    wbil = W_bil.reshape(N_H, N_H)
    bias2 = gcn_bias.reshape(1, N_H)
    al2 = alpha.reshape(1, 1)
    pa2 = prelu_a.reshape(1, 1)
    bb2 = b_bil.reshape(1, 1)

    k_idx = np.arange(16)[:, None]
    l_idx = np.arange(16 * N_H)[None, :]
    q = jnp.asarray((l_idx // N_H == k_idx).astype(np.float32))

    l1, l2, fix = pl.pallas_call(
        _body,
        grid=(G,),
        in_specs=[
            pl.BlockSpec((4 * BB, N_IN), lambda i: (i, 0)),
            pl.BlockSpec((BB, N * N), lambda i: (i, 0)),
            pl.BlockSpec((BB, N * N_H), lambda i: (i, 0)),
            pl.BlockSpec((N_IN, N_H), lambda i: (0, 0)),
            pl.BlockSpec((16, 16 * N_H), lambda i: (0, 0)),
            pl.BlockSpec((N_H, N_H), lambda i: (0, 0)),
            pl.BlockSpec((1, N_H), lambda i: (0, 0)),
            pl.BlockSpec((1, 1), lambda i: (0, 0)),
            pl.BlockSpec((1, 1), lambda i: (0, 0)),
            pl.BlockSpec((1, 1), lambda i: (0, 0)),
        ],
        out_specs=(
            pl.BlockSpec((BB, 1), lambda i: (i, 0)),
            pl.BlockSpec((BB, 1), lambda i: (i, 0)),
            pl.BlockSpec((1, 1), lambda i: (0, 0)),
        ),
        out_shape=(
            jax.ShapeDtypeStruct((B, 1), jnp.float32),
            jax.ShapeDtypeStruct((B, 1), jnp.float32),
            jax.ShapeDtypeStruct((1, 1), jnp.float32),
        ),
        scratch_shapes=[
            pltpu.VMEM((1, N_H), jnp.float32),
            pltpu.VMEM((1, N_H), jnp.float32),
        ],
        compiler_params=pltpu.CompilerParams(
            dimension_semantics=("arbitrary",),
        ),
    )(x, adj2, gns, W_fc, q, wbil, bias2, al2, pa2, bb2)

    l2 = l2.at[0, 0].set(fix[0, 0])
    return jnp.concatenate([l1, l2], axis=0)


# speedup vs baseline: 2.0072x; 1.0489x over previous
import numpy as np

import jax
import jax.numpy as jnp
from jax.experimental import pallas as pl
from jax.experimental.pallas import tpu as pltpu


def _body(x_ref, adj_ref, gns_ref, xh_ref, ah_ref, gh_ref, wfc_ref, q_ref,
          wbil_ref, bias_ref, al_ref, pa_ref, bb_ref,
          out_ref, gprev_ref):
    i = pl.program_id(0)
    nblk = pl.num_programs(0)

    n_h = wbil_ref.shape[0]
    a = pa_ref[0, 0]
    al = al_ref[0, 0]
    bb = bb_ref[0, 0]
    bias = bias_ref[...]

    fts_flat = jnp.dot(x_ref[...], wfc_ref[...],
                       preferred_element_type=jnp.float32)
    bbk = fts_flat.shape[0] // 4
    fts3 = fts_flat.reshape(bbk, 4, n_h)
    f0 = fts3[:, 0, :]
    f1 = fts3[:, 1, :]
    f2 = fts3[:, 2, :]
    f3 = fts3[:, 3, :]

    abig = jnp.dot(adj_ref[...], q_ref[...],
                   preferred_element_type=jnp.float32)

    def node(r):
        o = (abig[:, 256 * r + 0:256 * r + 64] * f0
             + abig[:, 256 * r + 64:256 * r + 128] * f1
             + abig[:, 256 * r + 128:256 * r + 192] * f2
             + abig[:, 256 * r + 192:256 * r + 256] * f3) + bias
        return jnp.where(o >= 0.0, o, a * o)

    c = (node(0) + node(1) + node(2)) * (1.0 / 3.0)
    hmv = node(3)

    gns = gns_ref[...]
    negc = (gns[:, 0:n_h] + gns[:, n_h:2 * n_h]
            + gns[:, 2 * n_h:3 * n_h]) * (1.0 / 3.0)

    g = al * c + (1.0 - al) * negc
    t = jnp.dot(hmv, wbil_ref[...], preferred_element_type=jnp.float32)

    @pl.when(i == 0)
    def _():
        fh = jnp.dot(xh_ref[...], wfc_ref[...],
                     preferred_element_type=jnp.float32)
        ah = ah_ref[...]
        ch = jnp.zeros((1, n_h), jnp.float32)
        for r in range(3):
            oh = (ah[7, 4 * r + 0] * fh[28:29, :]
                  + ah[7, 4 * r + 1] * fh[29:30, :]
                  + ah[7, 4 * r + 2] * fh[30:31, :]
                  + ah[7, 4 * r + 3] * fh[31:32, :]) + bias
            ch = ch + jnp.where(oh >= 0.0, oh, a * oh)
        ch = ch * (1.0 / 3.0)
        gh = gh_ref[...]
        nh = (gh[7:8, 0:n_h] + gh[7:8, n_h:2 * n_h]
              + gh[7:8, 2 * n_h:3 * n_h]) * (1.0 / 3.0)
        gprev_ref[...] = al * ch + (1.0 - al) * nh

    out_ref[0] = jnp.sum(t * g, axis=1, keepdims=True) + bb

    g_roll = pltpu.roll(g, 1, axis=0)
    row0 = jax.lax.broadcasted_iota(jnp.int32, g.shape, 0) == 0
    g_sh = jnp.where(row0, gprev_ref[...], g_roll)
    out_ref[1] = jnp.sum(t * g_sh, axis=1, keepdims=True) + bb

    @pl.when(i != nblk - 2)
    def _():
        gprev_ref[...] = g[bbk - 1:bbk, :]

    @pl.when(i == nblk - 2)
    def _():
        gprev_ref[...] = g[bbk - 2:bbk - 1, :]


def kernel(seq1, adj, glob_neg_seq, glob_neg_adj, alpha, W_fc, gcn_bias,
           prelu_a, W_bil, b_bil):
    B, N, N_IN = seq1.shape
    N_H = W_fc.shape[1]
    BB = 2000
    assert B % BB == 0
    G = B // BB

    x = seq1.reshape(B * N, N_IN)
    adj2 = adj.reshape(B, N * N)
    gns = glob_neg_seq.reshape(B, N * N_H)
    wbil = W_bil.reshape(N_H, N_H)
    bias2 = gcn_bias.reshape(1, N_H)
    al2 = alpha.reshape(1, 1)
    pa2 = prelu_a.reshape(1, 1)
    bb2 = b_bil.reshape(1, 1)

    k_idx = np.arange(16)[:, None]
    l_idx = np.arange(16 * N_H)[None, :]
    q = jnp.asarray((l_idx // N_H == k_idx).astype(np.float32))

    def blk(s):
        return jnp.where(s == G - 1, 0, s + 1)

    out3 = pl.pallas_call(
        _body,
        grid=(G,),
        in_specs=[
            pl.BlockSpec((4 * BB, N_IN), lambda s: (blk(s), 0)),
            pl.BlockSpec((BB, N * N), lambda s: (blk(s), 0)),
            pl.BlockSpec((BB, N * N_H), lambda s: (blk(s), 0)),
            pl.BlockSpec((32, N_IN), lambda s: (BB // 8 - 1, 0)),
            pl.BlockSpec((8, N * N), lambda s: (BB // 8 - 1, 0)),
            pl.BlockSpec((8, N * N_H), lambda s: (BB // 8 - 1, 0)),
            pl.BlockSpec((N_IN, N_H), lambda s: (0, 0)),
            pl.BlockSpec((16, 16 * N_H), lambda s: (0, 0)),
            pl.BlockSpec((N_H, N_H), lambda s: (0, 0)),
            pl.BlockSpec((1, N_H), lambda s: (0, 0)),
            pl.BlockSpec((1, 1), lambda s: (0, 0)),
            pl.BlockSpec((1, 1), lambda s: (0, 0)),
            pl.BlockSpec((1, 1), lambda s: (0, 0)),
        ],
        out_specs=pl.BlockSpec((2, BB, 1), lambda s: (0, blk(s), 0)),
        out_shape=jax.ShapeDtypeStruct((2, B, 1), jnp.float32),
        scratch_shapes=[
            pltpu.VMEM((1, N_H), jnp.float32),
        ],
        compiler_params=pltpu.CompilerParams(
            dimension_semantics=("arbitrary",),
        ),
    )(x, adj2, gns, x, adj2, gns, W_fc, q, wbil, bias2, al2, pa2, bb2)

    return out3.reshape(2 * B, 1)


# speedup vs baseline: 2.2164x; 1.1042x over previous
import numpy as np

import jax
import jax.numpy as jnp
from jax.experimental import pallas as pl
from jax.experimental.pallas import tpu as pltpu


def _body(x_ref, adj_ref, gns_ref, xh_ref, ah_ref, gh_ref, wfc_ref, q_ref,
          wbil_ref, bias_ref, al_ref, pa_ref, bb_ref,
          out_ref, gprev_ref):
    i = pl.program_id(0)
    nblk = pl.num_programs(0)

    n_h = wbil_ref.shape[0]
    a = pa_ref[0, 0]
    al = al_ref[0, 0]
    bb = bb_ref[0, 0]
    bias = bias_ref[...]

    fts_flat = jnp.dot(x_ref[...], wfc_ref[...],
                       preferred_element_type=jnp.float32)
    bbk = fts_flat.shape[0] // 4
    fts3 = fts_flat.reshape(bbk, 4, n_h)
    f0 = fts3[:, 0, :]
    f1 = fts3[:, 1, :]
    f2 = fts3[:, 2, :]
    f3 = fts3[:, 3, :]

    abig = jnp.dot(adj_ref[...], q_ref[...],
                   preferred_element_type=jnp.float32)

    def node(r):
        o = (abig[:, 256 * r + 0:256 * r + 64] * f0
             + abig[:, 256 * r + 64:256 * r + 128] * f1
             + abig[:, 256 * r + 128:256 * r + 192] * f2
             + abig[:, 256 * r + 192:256 * r + 256] * f3) + bias
        return jnp.where(o >= 0.0, o, a * o)

    c = (node(0) + node(1) + node(2)) * (1.0 / 3.0)
    hmv = node(3)

    gns = gns_ref[...]
    negc = (gns[:, 0:n_h] + gns[:, n_h:2 * n_h]
            + gns[:, 2 * n_h:3 * n_h]) * (1.0 / 3.0)

    g = al * c + (1.0 - al) * negc
    t = jnp.dot(hmv, wbil_ref[...], preferred_element_type=jnp.float32)

    @pl.when(i == 0)
    def _():
        fh = jnp.dot(xh_ref[...], wfc_ref[...],
                     preferred_element_type=jnp.float32)
        ah = ah_ref[...]
        ch = jnp.zeros((1, n_h), jnp.float32)
        for r in range(3):
            oh = (ah[7, 4 * r + 0] * fh[28:29, :]
                  + ah[7, 4 * r + 1] * fh[29:30, :]
                  + ah[7, 4 * r + 2] * fh[30:31, :]
                  + ah[7, 4 * r + 3] * fh[31:32, :]) + bias
            ch = ch + jnp.where(oh >= 0.0, oh, a * oh)
        ch = ch * (1.0 / 3.0)
        gh = gh_ref[...]
        nh = (gh[7:8, 0:n_h] + gh[7:8, n_h:2 * n_h]
              + gh[7:8, 2 * n_h:3 * n_h]) * (1.0 / 3.0)
        gprev_ref[...] = al * ch + (1.0 - al) * nh

    ones_row = jnp.ones((1, n_h), jnp.float32)
    dnum = (((1,), (1,)), ((), ()))

    out_ref[0, 0:1, :] = jax.lax.dot_general(
        ones_row, t * g, dnum,
        preferred_element_type=jnp.float32) + bb

    g_roll = pltpu.roll(g, 1, axis=0)
    row0 = jax.lax.broadcasted_iota(jnp.int32, g.shape, 0) == 0
    g_sh = jnp.where(row0, gprev_ref[...], g_roll)
    out_ref[0, 1:2, :] = jax.lax.dot_general(
        ones_row, t * g_sh, dnum,
        preferred_element_type=jnp.float32) + bb

    @pl.when(i != nblk - 2)
    def _():
        gprev_ref[...] = g[bbk - 1:bbk, :]

    @pl.when(i == nblk - 2)
    def _():
        gprev_ref[...] = g[bbk - 2:bbk - 1, :]


def kernel(seq1, adj, glob_neg_seq, glob_neg_adj, alpha, W_fc, gcn_bias,
           prelu_a, W_bil, b_bil):
    B, N, N_IN = seq1.shape
    N_H = W_fc.shape[1]
    BB = 2000
    assert B % BB == 0
    G = B // BB

    x = seq1.reshape(B * N, N_IN)
    adj2 = adj.reshape(B, N * N)
    gns = glob_neg_seq.reshape(B, N * N_H)
    wbil = W_bil.reshape(N_H, N_H)
    bias2 = gcn_bias.reshape(1, N_H)
    al2 = alpha.reshape(1, 1)
    pa2 = prelu_a.reshape(1, 1)
    bb2 = b_bil.reshape(1, 1)

    k_idx = np.arange(16)[:, None]
    l_idx = np.arange(16 * N_H)[None, :]
    q = jnp.asarray((l_idx // N_H == k_idx).astype(np.float32))

    def blk(s):
        return jnp.where(s == G - 1, 0, s + 1)

    out3 = pl.pallas_call(
        _body,
        grid=(G,),
        in_specs=[
            pl.BlockSpec((4 * BB, N_IN), lambda s: (blk(s), 0)),
            pl.BlockSpec((BB, N * N), lambda s: (blk(s), 0)),
            pl.BlockSpec((BB, N * N_H), lambda s: (blk(s), 0)),
            pl.BlockSpec((32, N_IN), lambda s: (BB // 8 - 1, 0)),
            pl.BlockSpec((8, N * N), lambda s: (BB // 8 - 1, 0)),
            pl.BlockSpec((8, N * N_H), lambda s: (BB // 8 - 1, 0)),
            pl.BlockSpec((N_IN, N_H), lambda s: (0, 0)),
            pl.BlockSpec((16, 16 * N_H), lambda s: (0, 0)),
            pl.BlockSpec((N_H, N_H), lambda s: (0, 0)),
            pl.BlockSpec((1, N_H), lambda s: (0, 0)),
            pl.BlockSpec((1, 1), lambda s: (0, 0)),
            pl.BlockSpec((1, 1), lambda s: (0, 0)),
            pl.BlockSpec((1, 1), lambda s: (0, 0)),
        ],
        out_specs=pl.BlockSpec((1, 2, BB), lambda s: (blk(s), 0, 0)),
        out_shape=jax.ShapeDtypeStruct((G, 2, BB), jnp.float32),
        scratch_shapes=[
            pltpu.VMEM((1, N_H), jnp.float32),
        ],
        compiler_params=pltpu.CompilerParams(
            dimension_semantics=("arbitrary",),
        ),
    )(x, adj2, gns, x, adj2, gns, W_fc, q, wbil, bias2, al2, pa2, bb2)

    return out3.transpose(1, 0, 2).reshape(2 * B, 1)


# speedup vs baseline: 2.4380x; 1.1000x over previous
import numpy as np

import jax
import jax.numpy as jnp
from jax.experimental import pallas as pl
from jax.experimental.pallas import tpu as pltpu


def _body(x_ref, adj_ref, gns_ref, xh_ref, ah_ref, gh_ref, wfc_ref, q_ref,
          ffold_ref, wbil_ref, bias_ref, al_ref, pa_ref, bb_ref,
          out_ref, gprev_ref):
    i = pl.program_id(0)
    nblk = pl.num_programs(0)

    n_h = wbil_ref.shape[0]
    a = pa_ref[0, 0]
    al = al_ref[0, 0]
    bb = bb_ref[0, 0]
    bias = bias_ref[...]

    fts_flat = jnp.dot(x_ref[...], wfc_ref[...],
                       preferred_element_type=jnp.float32)
    bbk = fts_flat.shape[0] // 4
    fts3 = fts_flat.reshape(bbk, 4, n_h)
    f0 = fts3[:, 0, :]
    f1 = fts3[:, 1, :]
    f2 = fts3[:, 2, :]
    f3 = fts3[:, 3, :]
    fpall = jnp.concatenate([f0, f1, f2, f3], axis=1)

    abig = jnp.dot(adj_ref[...], q_ref[...],
                   preferred_element_type=jnp.float32)

    ffold = ffold_ref[...]

    def node(r):
        s = abig[:, 256 * r:256 * (r + 1)] * fpall
        o = jnp.dot(s, ffold, preferred_element_type=jnp.float32) + bias
        return jnp.where(o >= 0.0, o, a * o)

    c = (node(0) + node(1) + node(2)) * (1.0 / 3.0)
    hmv = node(3)

    gns = gns_ref[...]
    negc = (gns[:, 0:n_h] + gns[:, n_h:2 * n_h]
            + gns[:, 2 * n_h:3 * n_h]) * (1.0 / 3.0)

    g = al * c + (1.0 - al) * negc
    t = jnp.dot(hmv, wbil_ref[...], preferred_element_type=jnp.float32)

    @pl.when(i == 0)
    def _():
        fh = jnp.dot(xh_ref[...], wfc_ref[...],
                     preferred_element_type=jnp.float32)
        ah = ah_ref[...]
        ch = jnp.zeros((1, n_h), jnp.float32)
        for r in range(3):
            oh = (ah[7, 4 * r + 0] * fh[28:29, :]
                  + ah[7, 4 * r + 1] * fh[29:30, :]
                  + ah[7, 4 * r + 2] * fh[30:31, :]
                  + ah[7, 4 * r + 3] * fh[31:32, :]) + bias
            ch = ch + jnp.where(oh >= 0.0, oh, a * oh)
        ch = ch * (1.0 / 3.0)
        gh = gh_ref[...]
        nh = (gh[7:8, 0:n_h] + gh[7:8, n_h:2 * n_h]
              + gh[7:8, 2 * n_h:3 * n_h]) * (1.0 / 3.0)
        gprev_ref[...] = al * ch + (1.0 - al) * nh

    ones_row = jnp.ones((1, n_h), jnp.float32)
    dnum = (((1,), (1,)), ((), ()))

    out_ref[0, 0:1, :] = jax.lax.dot_general(
        ones_row, t * g, dnum,
        preferred_element_type=jnp.float32) + bb

    g_roll = pltpu.roll(g, 1, axis=0)
    row0 = jax.lax.broadcasted_iota(jnp.int32, g.shape, 0) == 0
    g_sh = jnp.where(row0, gprev_ref[...], g_roll)
    out_ref[0, 1:2, :] = jax.lax.dot_general(
        ones_row, t * g_sh, dnum,
        preferred_element_type=jnp.float32) + bb

    @pl.when(i != nblk - 2)
    def _():
        gprev_ref[...] = g[bbk - 1:bbk, :]

    @pl.when(i == nblk - 2)
    def _():
        gprev_ref[...] = g[bbk - 2:bbk - 1, :]


def kernel(seq1, adj, glob_neg_seq, glob_neg_adj, alpha, W_fc, gcn_bias,
           prelu_a, W_bil, b_bil):
    B, N, N_IN = seq1.shape
    N_H = W_fc.shape[1]
    BB = 2000
    assert B % BB == 0
    G = B // BB

    x = seq1.reshape(B * N, N_IN)
    adj2 = adj.reshape(B, N * N)
    gns = glob_neg_seq.reshape(B, N * N_H)
    wbil = W_bil.reshape(N_H, N_H)
    bias2 = gcn_bias.reshape(1, N_H)
    al2 = alpha.reshape(1, 1)
    pa2 = prelu_a.reshape(1, 1)
    bb2 = b_bil.reshape(1, 1)

    k_idx = np.arange(16)[:, None]
    l_idx = np.arange(16 * N_H)[None, :]
    q = jnp.asarray((l_idx // N_H == k_idx).astype(np.float32))
    ffold = jnp.asarray(np.tile(np.eye(N_H, dtype=np.float32), (4, 1)))

    def blk(s):
        return jnp.where(s == G - 1, 0, s + 1)

    out3 = pl.pallas_call(
        _body,
        grid=(G,),
        in_specs=[
            pl.BlockSpec((4 * BB, N_IN), lambda s: (blk(s), 0)),
            pl.BlockSpec((BB, N * N), lambda s: (blk(s), 0)),
            pl.BlockSpec((BB, N * N_H), lambda s: (blk(s), 0)),
            pl.BlockSpec((32, N_IN), lambda s: (BB // 8 - 1, 0)),
            pl.BlockSpec((8, N * N), lambda s: (BB // 8 - 1, 0)),
            pl.BlockSpec((8, N * N_H), lambda s: (BB // 8 - 1, 0)),
            pl.BlockSpec((N_IN, N_H), lambda s: (0, 0)),
            pl.BlockSpec((16, 16 * N_H), lambda s: (0, 0)),
            pl.BlockSpec((4 * N_H, N_H), lambda s: (0, 0)),
            pl.BlockSpec((N_H, N_H), lambda s: (0, 0)),
            pl.BlockSpec((1, N_H), lambda s: (0, 0)),
            pl.BlockSpec((1, 1), lambda s: (0, 0)),
            pl.BlockSpec((1, 1), lambda s: (0, 0)),
            pl.BlockSpec((1, 1), lambda s: (0, 0)),
        ],
        out_specs=pl.BlockSpec((1, 2, BB), lambda s: (blk(s), 0, 0)),
        out_shape=jax.ShapeDtypeStruct((G, 2, BB), jnp.float32),
        scratch_shapes=[
            pltpu.VMEM((1, N_H), jnp.float32),
        ],
        compiler_params=pltpu.CompilerParams(
            dimension_semantics=("arbitrary",),
        ),
    )(x, adj2, gns, x, adj2, gns, W_fc, q, ffold, wbil, bias2, al2, pa2, bb2)

    return out3.transpose(1, 0, 2).reshape(2 * B, 1)
